# baseline (speedup 1.0000x reference)
import functools

import jax
import jax.numpy as jnp
from jax import lax
from jax.experimental import pallas as pl
from jax.experimental.pallas import tpu as pltpu
from jax.experimental.pallas import tpu_sc as plsc

N_DIM = 32
BATCH = 16384
NC = 2
NS = 16
NW = NC * NS
B_PER_W = BATCH // NW
IDX_CHUNK = 128
N_CHUNKS = B_PER_W // IDX_CHUNK
LANES = 16
N_GROUPS = B_PER_W // LANES


def _mf_body(u_hbm, i_hbm, ub_hbm, uvT_hbm, ib_hbm, ivT_hbm, gb_hbm, out_hbm,
             u_idx, i_idx, uval, ival, bu, bi, out_v, gv, sem):
    wid = lax.axis_index("s") * NC + lax.axis_index("c")
    base = wid * B_PER_W

    for c in range(N_CHUNKS):
        pltpu.sync_copy(u_hbm.at[pl.ds(base + c * IDX_CHUNK, IDX_CHUNK)],
                        u_idx.at[c])
        pltpu.sync_copy(i_hbm.at[pl.ds(base + c * IDX_CHUNK, IDX_CHUNK)],
                        i_idx.at[c])
    pltpu.sync_copy(gb_hbm, gv)
    gvec = gv[...]

    copies = []
    for c in range(N_CHUNKS):
        lo = c * IDX_CHUNK
        copies.append(pltpu.async_copy(
            ub_hbm.at[u_idx.at[c]], bu.at[pl.ds(lo, IDX_CHUNK)], sem))
        copies.append(pltpu.async_copy(
            ib_hbm.at[i_idx.at[c]], bi.at[pl.ds(lo, IDX_CHUNK)], sem))
    for cp in copies:
        cp.wait()

    def init_group(g, carry):
        row = g * LANES
        out_v[pl.ds(row, LANES)] = (
            bu[pl.ds(row, LANES)] + bi[pl.ds(row, LANES)] + gvec)
        return carry

    lax.fori_loop(0, N_GROUPS, init_group, 0)

    def fire(d):
        par = d % 2
        off = par * B_PER_W
        cps = []
        for c in range(N_CHUNKS):
            lo = off + c * IDX_CHUNK
            cps.append(pltpu.async_copy(
                uvT_hbm.at[pl.ds(d * 1000000, 1000000)].at[u_idx.at[c]],
                uval.at[pl.ds(lo, IDX_CHUNK)], sem))
            cps.append(pltpu.async_copy(
                ivT_hbm.at[pl.ds(d * 1000000, 1000000)].at[i_idx.at[c]],
                ival.at[pl.ds(lo, IDX_CHUNK)], sem))
        return cps

    def accumulate(d):
        off = (d % 2) * B_PER_W

        def acc_group(g, carry):
            row = g * LANES
            out_v[pl.ds(row, LANES)] += (
                uval[pl.ds(off + row, LANES)] * ival[pl.ds(off + row, LANES)])
            return carry

        lax.fori_loop(0, N_GROUPS, acc_group, 0)

    prev = fire(0)
    for d in range(1, N_DIM):
        nxt = fire(d)
        for cp in prev:
            cp.wait()
        accumulate(d - 1)
        prev = nxt
    for cp in prev:
        cp.wait()
    accumulate(N_DIM - 1)

    pltpu.sync_copy(out_v, out_hbm.at[pl.ds(base, B_PER_W)])


_mf = functools.partial(
    pl.kernel,
    mesh=plsc.VectorSubcoreMesh(core_axis_name="c", subcore_axis_name="s"),
    out_type=jax.ShapeDtypeStruct((BATCH,), jnp.float32),
    compiler_params=pltpu.CompilerParams(
        needs_layout_passes=False, use_tc_tiling_on_sc=False),
    scratch_types=[
        pltpu.VMEM((N_CHUNKS, IDX_CHUNK), jnp.int32),
        pltpu.VMEM((N_CHUNKS, IDX_CHUNK), jnp.int32),
        pltpu.VMEM((2 * B_PER_W,), jnp.float32),
        pltpu.VMEM((2 * B_PER_W,), jnp.float32),
        pltpu.VMEM((B_PER_W,), jnp.float32),
        pltpu.VMEM((B_PER_W,), jnp.float32),
        pltpu.VMEM((B_PER_W,), jnp.float32),
        pltpu.VMEM((LANES,), jnp.float32),
        pltpu.SemaphoreType.DMA,
    ],
)(_mf_body)


@jax.jit
def kernel(u, i, user_bias, user_vec, item_bias, item_vec, glob_bias):
    u = u.astype(jnp.int32)
    i = i.astype(jnp.int32)
    gb = jnp.broadcast_to(glob_bias.reshape(()), (LANES,))
    uv_flat = user_vec.T.reshape((N_DIM * 1000000,))
    iv_flat = item_vec.T.reshape((N_DIM * 1000000,))
    return _mf(u, i, user_bias, uv_flat, item_bias, iv_flat, gb)


# speedup vs baseline: 8.9176x; 8.9176x over previous
import functools

import jax
import jax.numpy as jnp
from jax import lax
from jax.experimental import pallas as pl
from jax.experimental.pallas import tpu as pltpu
from jax.experimental.pallas import tpu_sc as plsc

N_DIM = 32
N_ROWS = 1000000
BATCH = 16384
NC = 2
NS = 16
NW = NC * NS
B_PER_W = BATCH // NW
IDX_CHUNK = 128
N_CHUNKS = B_PER_W // IDX_CHUNK
LANES = 16
N_GROUPS = B_PER_W // LANES

TCOLS = 7813
TILE_W = 1024
SLAB = TCOLS * TILE_W
STAGE_WORDS = 4 * SLAB
JROW_MAX = (TCOLS - 1) * TILE_W + 128

COLS_PER_W = 245
CHUNK_TC = 16
CHUNK_W = CHUNK_TC * 128
N_STREAM_CHUNKS = 16
MAX_TB = TCOLS - 1 - CHUNK_TC
TAIL_BASE = (TCOLS - 1) * 128
TAIL_W = N_ROWS - TAIL_BASE


def _restage_body(uvT_hbm, ivT_hbm, ustage_hbm, istage_hbm,
                  buf0, buf1, lin0, lin1, tbuf,
                  rsem0, rsem1, wsem0, wsem1):
    wid = lax.axis_index("s") * NC + lax.axis_index("c")

    bufs = (buf0, buf1)
    lins = (lin0, lin1)
    rsems = (rsem0, rsem1)
    wsems = (wsem0, wsem1)

    def shuffle(buf, lin):
        def move(g, carry):
            m = g * LANES
            for s in range(8):
                lin[pl.ds(s * CHUNK_W + m, LANES)] = buf[s, pl.ds(m, LANES)]
            return carry

        lax.fori_loop(0, CHUNK_W // LANES, move, 0)

    for table, stage in ((uvT_hbm, ustage_hbm), (ivT_hbm, istage_hbm)):
        for ti in range(4):
            def colw_of(c):
                tb = jnp.minimum(COLS_PER_W * wid + CHUNK_TC * c, MAX_TB)
                return pl.multiple_of(tb * 128, 128)

            def read(c, p, table=table, ti=ti):
                pltpu.async_copy(
                    table.at[pl.ds(8 * ti, 8), pl.ds(colw_of(c), CHUNK_W)],
                    bufs[p], rsems[p])

            def process(c, p, g, table=table, ti=ti, stage=stage):
                pltpu.make_async_copy(
                    stage.at[pl.ds(0, 8 * CHUNK_W)], lins[p],
                    rsems[p]).wait()

                @pl.when(g >= 1)
                def _():
                    pltpu.make_async_copy(
                        stage.at[pl.ds(0, 8 * CHUNK_W)], lins[p],
                        wsems[p]).wait()

                shuffle(bufs[p], lins[p])
                colw = colw_of(c)
                for s in range(8):
                    pltpu.async_copy(
                        lins[p].at[pl.ds(s * CHUNK_W, CHUNK_W)],
                        stage.at[pl.ds((8 * ti + s) * N_ROWS, N_ROWS)]
                             .at[pl.ds(colw, CHUNK_W)],
                        wsems[p])

            read(0, 0)

            def body(g, carry):
                read(2 * g + 1, 1)
                process(2 * g, 0, g)
                read(2 * g + 2, 0)
                process(2 * g + 1, 1, g)
                return carry

            lax.fori_loop(0, N_STREAM_CHUNKS // 2, body, 0)
            pltpu.make_async_copy(
                stage.at[pl.ds(0, 8 * CHUNK_W)], lins[0], rsems[0]).wait()
            for p in range(2):
                pltpu.make_async_copy(
                    stage.at[pl.ds(0, 8 * CHUNK_W)],
                    lins[p], wsems[p]).wait()

    @pl.when(wid == NW - 1)
    def _():
        for table, stage in ((uvT_hbm, ustage_hbm), (ivT_hbm, istage_hbm)):
            for ti in range(4):
                pltpu.sync_copy(
                    table.at[pl.ds(8 * ti, 8), pl.ds(TAIL_BASE, TAIL_W)],
                    tbuf)
                for s in range(8):
                    for g in range(TAIL_W // LANES):
                        m = g * LANES
                        lin0[pl.ds(s * TAIL_W + m, LANES)] = (
                            tbuf[s, pl.ds(m, LANES)])
                    pltpu.sync_copy(
                        lin0.at[pl.ds(s * TAIL_W, TAIL_W)],
                        stage.at[pl.ds((8 * ti + s) * N_ROWS + TAIL_BASE,
                                       TAIL_W)])


_restage = functools.partial(
    pl.kernel,
    mesh=plsc.VectorSubcoreMesh(core_axis_name="c", subcore_axis_name="s"),
    out_type=(jax.ShapeDtypeStruct((N_DIM * N_ROWS,), jnp.float32),
              jax.ShapeDtypeStruct((N_DIM * N_ROWS,), jnp.float32)),
    compiler_params=pltpu.CompilerParams(
        needs_layout_passes=False, use_tc_tiling_on_sc=True),
    scratch_types=[
        pltpu.VMEM((8, CHUNK_W), jnp.float32),
        pltpu.VMEM((8, CHUNK_W), jnp.float32),
        pltpu.VMEM((8 * CHUNK_W,), jnp.float32),
        pltpu.VMEM((8 * CHUNK_W,), jnp.float32),
        pltpu.VMEM((8, TAIL_W), jnp.float32),
        pltpu.SemaphoreType.DMA,
        pltpu.SemaphoreType.DMA,
        pltpu.SemaphoreType.DMA,
        pltpu.SemaphoreType.DMA,
    ],
)(_restage_body)


def _mf_body(u_hbm, i_hbm, ub_hbm, ustage_hbm, ib_hbm, istage_hbm, gb_hbm,
             out_hbm, u_idx, i_idx, uval, ival, bu, bi, out_v, gv, sem):
    wid = lax.axis_index("s") * NC + lax.axis_index("c")
    base = wid * B_PER_W

    for c in range(N_CHUNKS):
        pltpu.sync_copy(u_hbm.at[pl.ds(base + c * IDX_CHUNK, IDX_CHUNK)],
                        u_idx.at[c])
        pltpu.sync_copy(i_hbm.at[pl.ds(base + c * IDX_CHUNK, IDX_CHUNK)],
                        i_idx.at[c])
    pltpu.sync_copy(gb_hbm, gv)
    gvec = gv[...]

    copies = []
    for c in range(N_CHUNKS):
        lo = c * IDX_CHUNK
        copies.append(pltpu.async_copy(
            ub_hbm.at[u_idx.at[c]], bu.at[pl.ds(lo, IDX_CHUNK)], sem))
        copies.append(pltpu.async_copy(
            ib_hbm.at[i_idx.at[c]], bi.at[pl.ds(lo, IDX_CHUNK)], sem))
    for cp in copies:
        cp.wait()

    def init_group(g, carry):
        row = g * LANES
        out_v[pl.ds(row, LANES)] = (
            bu[pl.ds(row, LANES)] + bi[pl.ds(row, LANES)] + gvec)
        return carry

    lax.fori_loop(0, N_GROUPS, init_group, 0)

    def fire(d):
        par = d % 2
        off = par * B_PER_W
        cps = []
        for c in range(N_CHUNKS):
            lo = off + c * IDX_CHUNK
            cps.append(pltpu.async_copy(
                ustage_hbm.at[pl.ds(d * N_ROWS, N_ROWS)].at[u_idx.at[c]],
                uval.at[pl.ds(lo, IDX_CHUNK)], sem))
            cps.append(pltpu.async_copy(
                istage_hbm.at[pl.ds(d * N_ROWS, N_ROWS)].at[i_idx.at[c]],
                ival.at[pl.ds(lo, IDX_CHUNK)], sem))
        return cps

    def accumulate(d):
        off = (d % 2) * B_PER_W

        def acc_group(g, carry):
            row = g * LANES
            out_v[pl.ds(row, LANES)] += (
                uval[pl.ds(off + row, LANES)] * ival[pl.ds(off + row, LANES)])
            return carry

        lax.fori_loop(0, N_GROUPS, acc_group, 0)

    prev = fire(0)
    for d in range(1, N_DIM):
        nxt = fire(d)
        for cp in prev:
            cp.wait()
        accumulate(d - 1)
        prev = nxt
    for cp in prev:
        cp.wait()
    accumulate(N_DIM - 1)

    pltpu.sync_copy(out_v, out_hbm.at[pl.ds(base, B_PER_W)])


_mf = functools.partial(
    pl.kernel,
    mesh=plsc.VectorSubcoreMesh(core_axis_name="c", subcore_axis_name="s"),
    out_type=jax.ShapeDtypeStruct((BATCH,), jnp.float32),
    compiler_params=pltpu.CompilerParams(
        needs_layout_passes=False, use_tc_tiling_on_sc=False),
    scratch_types=[
        pltpu.VMEM((N_CHUNKS, IDX_CHUNK), jnp.int32),
        pltpu.VMEM((N_CHUNKS, IDX_CHUNK), jnp.int32),
        pltpu.VMEM((2 * B_PER_W,), jnp.float32),
        pltpu.VMEM((2 * B_PER_W,), jnp.float32),
        pltpu.VMEM((B_PER_W,), jnp.float32),
        pltpu.VMEM((B_PER_W,), jnp.float32),
        pltpu.VMEM((B_PER_W,), jnp.float32),
        pltpu.VMEM((LANES,), jnp.float32),
        pltpu.SemaphoreType.DMA,
    ],
)(_mf_body)


@jax.jit
def kernel(u, i, user_bias, user_vec, item_bias, item_vec, glob_bias):
    u = u.astype(jnp.int32)
    i = i.astype(jnp.int32)
    gb = jnp.broadcast_to(glob_bias.reshape(()), (LANES,))
    ustage, istage = _restage(user_vec.T, item_vec.T)
    return _mf(u, i, user_bias, ustage, item_bias, istage, gb)


# speedup vs baseline: 8.9731x; 1.0062x over previous
import functools

import jax
import jax.numpy as jnp
from jax import lax
from jax.experimental import pallas as pl
from jax.experimental.pallas import tpu as pltpu
from jax.experimental.pallas import tpu_sc as plsc

N_DIM = 32
N_ROWS = 1000000
BATCH = 16384
NC = 2
NS = 16
NW = NC * NS
B_PER_W = BATCH // NW
IDX_CHUNK = 128
N_CHUNKS = B_PER_W // IDX_CHUNK
LANES = 16
N_GROUPS = B_PER_W // LANES

TCOLS = 7813
TILE_W = 1024
SLAB = TCOLS * TILE_W
STAGE_WORDS = 4 * SLAB
JROW_MAX = (TCOLS - 1) * TILE_W + 128

COLS_PER_W = 245
CHUNK_TC = 16
CHUNK_W = CHUNK_TC * 128
N_STREAM_CHUNKS = 16
MAX_TB = TCOLS - 1 - CHUNK_TC
TAIL_BASE = (TCOLS - 1) * 128
TAIL_W = N_ROWS - TAIL_BASE


def _restage_body(uvT_hbm, ivT_hbm, ustage_hbm, istage_hbm,
                  buf0, buf1, lin0, lin1, tbuf,
                  rsem0, rsem1, wsem0, wsem1):
    wid = lax.axis_index("s") * NC + lax.axis_index("c")

    bufs = (buf0, buf1)
    lins = (lin0, lin1)
    rsems = (rsem0, rsem1)
    wsems = (wsem0, wsem1)

    def shuffle(buf, lin):
        def move(g, carry):
            m = g * (4 * LANES)
            for k in range(4):
                mk = m + k * LANES
                for s in range(8):
                    lin[pl.ds(s * CHUNK_W + mk, LANES)] = (
                        buf[s, pl.ds(mk, LANES)])
            return carry

        lax.fori_loop(0, CHUNK_W // (4 * LANES), move, 0)

    for table, stage in ((uvT_hbm, ustage_hbm), (ivT_hbm, istage_hbm)):
        for ti in range(4):
            def colw_of(c):
                tb = jnp.minimum(COLS_PER_W * wid + CHUNK_TC * c, MAX_TB)
                return pl.multiple_of(tb * 128, 128)

            def read(c, p, table=table, ti=ti):
                pltpu.async_copy(
                    table.at[pl.ds(8 * ti, 8), pl.ds(colw_of(c), CHUNK_W)],
                    bufs[p], rsems[p])

            def process(c, p, g, table=table, ti=ti, stage=stage):
                pltpu.make_async_copy(
                    stage.at[pl.ds(0, 8 * CHUNK_W)], lins[p],
                    rsems[p]).wait()

                @pl.when(g >= 1)
                def _():
                    pltpu.make_async_copy(
                        stage.at[pl.ds(0, 8 * CHUNK_W)], lins[p],
                        wsems[p]).wait()

                shuffle(bufs[p], lins[p])
                colw = colw_of(c)
                for s in range(8):
                    pltpu.async_copy(
                        lins[p].at[pl.ds(s * CHUNK_W, CHUNK_W)],
                        stage.at[pl.ds((8 * ti + s) * N_ROWS, N_ROWS)]
                             .at[pl.ds(colw, CHUNK_W)],
                        wsems[p])

            read(0, 0)

            def body(g, carry):
                read(2 * g + 1, 1)
                process(2 * g, 0, g)
                read(2 * g + 2, 0)
                process(2 * g + 1, 1, g)
                return carry

            lax.fori_loop(0, N_STREAM_CHUNKS // 2, body, 0)
            pltpu.make_async_copy(
                stage.at[pl.ds(0, 8 * CHUNK_W)], lins[0], rsems[0]).wait()
            for p in range(2):
                pltpu.make_async_copy(
                    stage.at[pl.ds(0, 8 * CHUNK_W)],
                    lins[p], wsems[p]).wait()

    @pl.when(wid == NW - 1)
    def _():
        for table, stage in ((uvT_hbm, ustage_hbm), (ivT_hbm, istage_hbm)):
            for ti in range(4):
                pltpu.sync_copy(
                    table.at[pl.ds(8 * ti, 8), pl.ds(TAIL_BASE, TAIL_W)],
                    tbuf)
                for s in range(8):
                    for g in range(TAIL_W // LANES):
                        m = g * LANES
                        lin0[pl.ds(s * TAIL_W + m, LANES)] = (
                            tbuf[s, pl.ds(m, LANES)])
                    pltpu.sync_copy(
                        lin0.at[pl.ds(s * TAIL_W, TAIL_W)],
                        stage.at[pl.ds((8 * ti + s) * N_ROWS + TAIL_BASE,
                                       TAIL_W)])


_restage = functools.partial(
    pl.kernel,
    mesh=plsc.VectorSubcoreMesh(core_axis_name="c", subcore_axis_name="s"),
    out_type=(jax.ShapeDtypeStruct((N_DIM * N_ROWS,), jnp.float32),
              jax.ShapeDtypeStruct((N_DIM * N_ROWS,), jnp.float32)),
    compiler_params=pltpu.CompilerParams(
        needs_layout_passes=False, use_tc_tiling_on_sc=True),
    scratch_types=[
        pltpu.VMEM((8, CHUNK_W), jnp.float32),
        pltpu.VMEM((8, CHUNK_W), jnp.float32),
        pltpu.VMEM((8 * CHUNK_W,), jnp.float32),
        pltpu.VMEM((8 * CHUNK_W,), jnp.float32),
        pltpu.VMEM((8, TAIL_W), jnp.float32),
        pltpu.SemaphoreType.DMA,
        pltpu.SemaphoreType.DMA,
        pltpu.SemaphoreType.DMA,
        pltpu.SemaphoreType.DMA,
    ],
)(_restage_body)


def _mf_body(u_hbm, i_hbm, ub_hbm, ustage_hbm, ib_hbm, istage_hbm, gb_hbm,
             out_hbm, u_idx, i_idx, uval, ival, bu, bi, out_v, gv, sem):
    wid = lax.axis_index("s") * NC + lax.axis_index("c")
    base = wid * B_PER_W

    for c in range(N_CHUNKS):
        pltpu.sync_copy(u_hbm.at[pl.ds(base + c * IDX_CHUNK, IDX_CHUNK)],
                        u_idx.at[c])
        pltpu.sync_copy(i_hbm.at[pl.ds(base + c * IDX_CHUNK, IDX_CHUNK)],
                        i_idx.at[c])
    pltpu.sync_copy(gb_hbm, gv)
    gvec = gv[...]

    copies = []
    for c in range(N_CHUNKS):
        lo = c * IDX_CHUNK
        copies.append(pltpu.async_copy(
            ub_hbm.at[u_idx.at[c]], bu.at[pl.ds(lo, IDX_CHUNK)], sem))
        copies.append(pltpu.async_copy(
            ib_hbm.at[i_idx.at[c]], bi.at[pl.ds(lo, IDX_CHUNK)], sem))
    for cp in copies:
        cp.wait()

    def init_group(g, carry):
        row = g * LANES
        out_v[pl.ds(row, LANES)] = (
            bu[pl.ds(row, LANES)] + bi[pl.ds(row, LANES)] + gvec)
        return carry

    lax.fori_loop(0, N_GROUPS, init_group, 0)

    def fire(d):
        par = d % 2
        off = par * B_PER_W
        cps = []
        for c in range(N_CHUNKS):
            lo = off + c * IDX_CHUNK
            cps.append(pltpu.async_copy(
                ustage_hbm.at[pl.ds(d * N_ROWS, N_ROWS)].at[u_idx.at[c]],
                uval.at[pl.ds(lo, IDX_CHUNK)], sem))
            cps.append(pltpu.async_copy(
                istage_hbm.at[pl.ds(d * N_ROWS, N_ROWS)].at[i_idx.at[c]],
                ival.at[pl.ds(lo, IDX_CHUNK)], sem))
        return cps

    def accumulate(d):
        off = (d % 2) * B_PER_W

        def acc_group(g, carry):
            row = g * LANES
            out_v[pl.ds(row, LANES)] += (
                uval[pl.ds(off + row, LANES)] * ival[pl.ds(off + row, LANES)])
            return carry

        lax.fori_loop(0, N_GROUPS, acc_group, 0)

    prev = fire(0)
    for d in range(1, N_DIM):
        nxt = fire(d)
        for cp in prev:
            cp.wait()
        accumulate(d - 1)
        prev = nxt
    for cp in prev:
        cp.wait()
    accumulate(N_DIM - 1)

    pltpu.sync_copy(out_v, out_hbm.at[pl.ds(base, B_PER_W)])


_mf = functools.partial(
    pl.kernel,
    mesh=plsc.VectorSubcoreMesh(core_axis_name="c", subcore_axis_name="s"),
    out_type=jax.ShapeDtypeStruct((BATCH,), jnp.float32),
    compiler_params=pltpu.CompilerParams(
        needs_layout_passes=False, use_tc_tiling_on_sc=False),
    scratch_types=[
        pltpu.VMEM((N_CHUNKS, IDX_CHUNK), jnp.int32),
        pltpu.VMEM((N_CHUNKS, IDX_CHUNK), jnp.int32),
        pltpu.VMEM((2 * B_PER_W,), jnp.float32),
        pltpu.VMEM((2 * B_PER_W,), jnp.float32),
        pltpu.VMEM((B_PER_W,), jnp.float32),
        pltpu.VMEM((B_PER_W,), jnp.float32),
        pltpu.VMEM((B_PER_W,), jnp.float32),
        pltpu.VMEM((LANES,), jnp.float32),
        pltpu.SemaphoreType.DMA,
    ],
)(_mf_body)


@jax.jit
def kernel(u, i, user_bias, user_vec, item_bias, item_vec, glob_bias):
    u = u.astype(jnp.int32)
    i = i.astype(jnp.int32)
    gb = jnp.broadcast_to(glob_bias.reshape(()), (LANES,))
    ustage, istage = _restage(user_vec.T, item_vec.T)
    return _mf(u, i, user_bias, ustage, item_bias, istage, gb)


# speedup vs baseline: 9.1213x; 1.0165x over previous
import functools

import jax
import jax.numpy as jnp
from jax import lax
from jax.experimental import pallas as pl
from jax.experimental.pallas import tpu as pltpu
from jax.experimental.pallas import tpu_sc as plsc

N_DIM = 32
N_ROWS = 1000000
BATCH = 16384
NC = 2
NS = 16
NW = NC * NS
B_PER_W = BATCH // NW
IDX_CHUNK = 128
N_CHUNKS = B_PER_W // IDX_CHUNK
LANES = 16
N_GROUPS = B_PER_W // LANES

TCOLS = 7813
TILE_W = 1024
SLAB = TCOLS * TILE_W
STAGE_WORDS = 4 * SLAB
JROW_MAX = (TCOLS - 1) * TILE_W + 128

COLS_PER_W = 245
CHUNK_TC = 8
CHUNK_W = CHUNK_TC * 128
N_STREAM_CHUNKS = 32
MAX_TB = TCOLS - 1 - CHUNK_TC
TAIL_BASE = (TCOLS - 1) * 128
TAIL_W = N_ROWS - TAIL_BASE


def _restage_body(uvT_hbm, ivT_hbm, ustage_hbm, istage_hbm,
                  buf00, buf01, buf10, buf11, lin00, lin01, lin10, lin11,
                  tbuf, rsem00, rsem01, rsem10, rsem11,
                  wsem00, wsem01, wsem10, wsem11):
    wid = lax.axis_index("s") * NC + lax.axis_index("c")

    tables = (uvT_hbm, ivT_hbm)
    stages = (ustage_hbm, istage_hbm)
    bufs_t = ((buf00, buf01), (buf10, buf11))
    lins_t = ((lin00, lin01), (lin10, lin11))
    rsems_t = ((rsem00, rsem01), (rsem10, rsem11))
    wsems_t = ((wsem00, wsem01), (wsem10, wsem11))

    def shuffle(buf, lin):
        def move(g, carry):
            m = g * (4 * LANES)
            for k in range(4):
                mk = m + k * LANES
                for s in range(8):
                    lin[pl.ds(s * CHUNK_W + mk, LANES)] = (
                        buf[s, pl.ds(mk, LANES)])
            return carry

        lax.fori_loop(0, CHUNK_W // (4 * LANES), move, 0)

    def colw_of(c):
        tb = jnp.minimum(COLS_PER_W * wid + CHUNK_TC * c, MAX_TB)
        return pl.multiple_of(tb * 128, 128)

    for ti in range(4):
        def read(t, c, p, ti=ti):
            pltpu.async_copy(
                tables[t].at[pl.ds(8 * ti, 8), pl.ds(colw_of(c), CHUNK_W)],
                bufs_t[t][p], rsems_t[t][p])

        def process(t, c, p, g, ti=ti):
            stage = stages[t]
            pltpu.make_async_copy(
                stage.at[pl.ds(0, 8 * CHUNK_W)], lins_t[t][p],
                rsems_t[t][p]).wait()

            @pl.when(g >= 1)
            def _():
                pltpu.make_async_copy(
                    stage.at[pl.ds(0, 8 * CHUNK_W)], lins_t[t][p],
                    wsems_t[t][p]).wait()

            shuffle(bufs_t[t][p], lins_t[t][p])
            colw = colw_of(c)
            for s in range(8):
                pltpu.async_copy(
                    lins_t[t][p].at[pl.ds(s * CHUNK_W, CHUNK_W)],
                    stage.at[pl.ds((8 * ti + s) * N_ROWS, N_ROWS)]
                         .at[pl.ds(colw, CHUNK_W)],
                    wsems_t[t][p])

        read(0, 0, 0)
        read(1, 0, 0)

        def body(g, carry):
            for t in range(2):
                read(t, 2 * g + 1, 1)
            for t in range(2):
                process(t, 2 * g, 0, g)
            for t in range(2):
                read(t, 2 * g + 2, 0)
            for t in range(2):
                process(t, 2 * g + 1, 1, g)
            return carry

        lax.fori_loop(0, N_STREAM_CHUNKS // 2, body, 0)
        for t in range(2):
            pltpu.make_async_copy(
                stages[t].at[pl.ds(0, 8 * CHUNK_W)],
                lins_t[t][0], rsems_t[t][0]).wait()
            for p in range(2):
                pltpu.make_async_copy(
                    stages[t].at[pl.ds(0, 8 * CHUNK_W)],
                    lins_t[t][p], wsems_t[t][p]).wait()

    @pl.when(wid == NW - 1)
    def _():
        for table, stage in ((uvT_hbm, ustage_hbm), (ivT_hbm, istage_hbm)):
            for ti in range(4):
                pltpu.sync_copy(
                    table.at[pl.ds(8 * ti, 8), pl.ds(TAIL_BASE, TAIL_W)],
                    tbuf)
                for s in range(8):
                    for g in range(TAIL_W // LANES):
                        m = g * LANES
                        lin00[pl.ds(s * TAIL_W + m, LANES)] = (
                            tbuf[s, pl.ds(m, LANES)])
                    pltpu.sync_copy(
                        lin00.at[pl.ds(s * TAIL_W, TAIL_W)],
                        stage.at[pl.ds((8 * ti + s) * N_ROWS + TAIL_BASE,
                                       TAIL_W)])


_restage = functools.partial(
    pl.kernel,
    mesh=plsc.VectorSubcoreMesh(core_axis_name="c", subcore_axis_name="s"),
    out_type=(jax.ShapeDtypeStruct((N_DIM * N_ROWS,), jnp.float32),
              jax.ShapeDtypeStruct((N_DIM * N_ROWS,), jnp.float32)),
    compiler_params=pltpu.CompilerParams(
        needs_layout_passes=False, use_tc_tiling_on_sc=True),
    scratch_types=(
        [pltpu.VMEM((8, CHUNK_W), jnp.float32)] * 4
        + [pltpu.VMEM((8 * CHUNK_W,), jnp.float32)] * 4
        + [pltpu.VMEM((8, TAIL_W), jnp.float32)]
        + [pltpu.SemaphoreType.DMA] * 8
    ),
)(_restage_body)


def _mf_body(u_hbm, i_hbm, ub_hbm, ustage_hbm, ib_hbm, istage_hbm, gb_hbm,
             out_hbm, u_idx, i_idx, uval, ival, bu, bi, out_v, gv, sem):
    wid = lax.axis_index("s") * NC + lax.axis_index("c")
    base = wid * B_PER_W

    for c in range(N_CHUNKS):
        pltpu.sync_copy(u_hbm.at[pl.ds(base + c * IDX_CHUNK, IDX_CHUNK)],
                        u_idx.at[c])
        pltpu.sync_copy(i_hbm.at[pl.ds(base + c * IDX_CHUNK, IDX_CHUNK)],
                        i_idx.at[c])
    pltpu.sync_copy(gb_hbm, gv)
    gvec = gv[...]

    copies = []
    for c in range(N_CHUNKS):
        lo = c * IDX_CHUNK
        copies.append(pltpu.async_copy(
            ub_hbm.at[u_idx.at[c]], bu.at[pl.ds(lo, IDX_CHUNK)], sem))
        copies.append(pltpu.async_copy(
            ib_hbm.at[i_idx.at[c]], bi.at[pl.ds(lo, IDX_CHUNK)], sem))
    for cp in copies:
        cp.wait()

    def init_group(g, carry):
        row = g * LANES
        out_v[pl.ds(row, LANES)] = (
            bu[pl.ds(row, LANES)] + bi[pl.ds(row, LANES)] + gvec)
        return carry

    lax.fori_loop(0, N_GROUPS, init_group, 0)

    def fire(d):
        par = d % 2
        off = par * B_PER_W
        cps = []
        for c in range(N_CHUNKS):
            lo = off + c * IDX_CHUNK
            cps.append(pltpu.async_copy(
                ustage_hbm.at[pl.ds(d * N_ROWS, N_ROWS)].at[u_idx.at[c]],
                uval.at[pl.ds(lo, IDX_CHUNK)], sem))
            cps.append(pltpu.async_copy(
                istage_hbm.at[pl.ds(d * N_ROWS, N_ROWS)].at[i_idx.at[c]],
                ival.at[pl.ds(lo, IDX_CHUNK)], sem))
        return cps

    def accumulate(d):
        off = (d % 2) * B_PER_W

        def acc_group(g, carry):
            row = g * LANES
            out_v[pl.ds(row, LANES)] += (
                uval[pl.ds(off + row, LANES)] * ival[pl.ds(off + row, LANES)])
            return carry

        lax.fori_loop(0, N_GROUPS, acc_group, 0)

    prev = fire(0)
    for d in range(1, N_DIM):
        nxt = fire(d)
        for cp in prev:
            cp.wait()
        accumulate(d - 1)
        prev = nxt
    for cp in prev:
        cp.wait()
    accumulate(N_DIM - 1)

    pltpu.sync_copy(out_v, out_hbm.at[pl.ds(base, B_PER_W)])


_mf = functools.partial(
    pl.kernel,
    mesh=plsc.VectorSubcoreMesh(core_axis_name="c", subcore_axis_name="s"),
    out_type=jax.ShapeDtypeStruct((BATCH,), jnp.float32),
    compiler_params=pltpu.CompilerParams(
        needs_layout_passes=False, use_tc_tiling_on_sc=False),
    scratch_types=[
        pltpu.VMEM((N_CHUNKS, IDX_CHUNK), jnp.int32),
        pltpu.VMEM((N_CHUNKS, IDX_CHUNK), jnp.int32),
        pltpu.VMEM((2 * B_PER_W,), jnp.float32),
        pltpu.VMEM((2 * B_PER_W,), jnp.float32),
        pltpu.VMEM((B_PER_W,), jnp.float32),
        pltpu.VMEM((B_PER_W,), jnp.float32),
        pltpu.VMEM((B_PER_W,), jnp.float32),
        pltpu.VMEM((LANES,), jnp.float32),
        pltpu.SemaphoreType.DMA,
    ],
)(_mf_body)


@jax.jit
def kernel(u, i, user_bias, user_vec, item_bias, item_vec, glob_bias):
    u = u.astype(jnp.int32)
    i = i.astype(jnp.int32)
    gb = jnp.broadcast_to(glob_bias.reshape(()), (LANES,))
    ustage, istage = _restage(user_vec.T, item_vec.T)
    return _mf(u, i, user_bias, ustage, item_bias, istage, gb)
